# baseline (speedup 1.0000x reference)
import functools
import math

import jax
import jax.numpy as jnp
from jax import lax
from jax.experimental import pallas as pl
from jax.experimental.pallas import tpu as pltpu
from jax.experimental.pallas import tpu_sc as plsc

NUM_SLOTS = 16
SLOT_DIM = 512
LM_HIDDEN = 2560
HIDDEN = 1024
TOP_M = 5
THR = 0.7
B = 32
P = 576
MROWS = 8
LANES = 16
NSEL = B * MROWS

_HIGH = jax.lax.Precision.HIGHEST
_INV_SQRT2 = 1.0 / math.sqrt(2.0)


def _gelu_exact(x):
    return 0.5 * x * (1.0 + lax.erf(x * _INV_SQRT2))


FB = 1


def _conf_kernel(x_ref, w1_ref, b1_ref, wct_ref, bc_ref, conf_ref):
    x = x_ref[...].reshape(FB * P, LM_HIDDEN)
    h = _gelu_exact(jnp.dot(x, w1_ref[...]) + b1_ref[...])
    conf = lax.dot_general(wct_ref[...], h, (((1,), (1,)), ((), ())))
    conf_ref[0] = conf + bc_ref[...]


def _combine(a, b):
    va, ga = a
    vb, gb = b
    keep_a = (va > vb) | ((va == vb) & (ga < gb))
    return jnp.where(keep_a, va, vb), jnp.where(keep_a, ga, gb)


def _make_sc_body(base, nframes):
    def _sc_body(conf_hbm, x_hbm, xsel_hbm, conf_v, idx_v, bcf_v, bci_v,
                 rows_v, sem):
        c = lax.axis_index("c")
        s = lax.axis_index("s")

        @pl.when((c * 16 + s) < nframes)
        def _():
            w = c * 16 + s
            b = base + w
            pltpu.sync_copy(conf_hbm.at[w], conf_v)

            lanes = lax.iota(jnp.int32, 16)
            nchunks = P // LANES
            gidx = [ci * LANES + lanes for ci in range(nchunks)]
            vals = [conf_v[pl.ds(ci * LANES, LANES)] for ci in range(nchunks)]

            picked = []
            for m in range(TOP_M):
                if picked:
                    last = picked[-1]
                    vals[:] = [jnp.where(g == last, -jnp.inf, v)
                               for g, v in zip(gidx, vals)]
                items = list(zip(vals, gidx))
                while len(items) > 1:
                    nxt = [_combine(items[i], items[i + 1])
                           for i in range(0, len(items) - 1, 2)]
                    if len(items) % 2:
                        nxt.append(items[-1])
                    items = nxt
                bv, bi = items[0]
                bcf_v[...] = bv
                bci_v[...] = bi
                splats = []
                for j in range(LANES):
                    sel = jnp.full((LANES,), j, jnp.int32)
                    splats.append((plsc.load_gather(bcf_v, [sel]),
                                   plsc.load_gather(bci_v, [sel])))
                while len(splats) > 1:
                    splats = [_combine(splats[i], splats[i + 1])
                              for i in range(0, len(splats), 2)]
                picked.append(splats[0][1])

            acc = jnp.zeros((LANES,), jnp.int32)
            for m, idx_m in enumerate(picked):
                acc = jnp.where(lanes == m, b * P + idx_m, acc)
            idx_v[...] = acc
            pltpu.async_copy(x_hbm.at[idx_v.at[pl.ds(0, MROWS)]], rows_v,
                             sem).wait()
            pltpu.sync_copy(rows_v, xsel_hbm.at[pl.ds(w * MROWS, MROWS)])

    return _sc_body


def _finish_kernel(xa_ref, xb_ref, w1_ref, b1_ref, w2c_ref, b2c_ref, slots_ref,
                   wg_ref, bg_ref, wp_ref, bp_ref, out_ref):
    xsel = jnp.concatenate([xa_ref[...], xb_ref[...]], axis=0)
    h = _gelu_exact(jnp.dot(xsel, w1_ref[...]) + b1_ref[...])
    cand = jnp.dot(h, w2c_ref[...]) + b2c_ref[...]

    cn = cand / (jnp.sqrt(jnp.sum(cand * cand, axis=1, keepdims=True)) + 1e-12)
    slots = slots_ref[...]
    sn = slots / (jnp.sqrt(jnp.sum(slots * slots, axis=1, keepdims=True)) + 1e-12)
    scores = lax.dot_general(cn, sn, (((1,), (1,)), ((), ())))

    best_score = jnp.max(scores, axis=1, keepdims=True)
    s_iota = lax.broadcasted_iota(jnp.int32, (NSEL, NUM_SLOTS), 1)
    idx = jnp.min(jnp.where(scores == best_score, s_iota, NUM_SLOTS),
                  axis=1, keepdims=True)

    onehot = jnp.where(s_iota == idx, 1.0, 0.0)
    old = jnp.dot(onehot, slots, precision=_HIGH)

    g = jax.nn.sigmoid(
        jnp.dot(jnp.concatenate([old, cand], axis=1), wg_ref[...]) + bg_ref[...])
    upd = g * cand + (1.0 - g) * old
    vals = jnp.where(best_score > THR, upd, old)

    r_iota = lax.broadcasted_iota(jnp.int32, (NSEL, 1), 0)
    bv, mv = r_iota // MROWS, r_iota % MROWS
    valid = mv < TOP_M
    rank = jnp.where(valid, 1 + bv * TOP_M + mv, -1)
    rmat = jnp.where((s_iota == idx) & valid, rank, -1)

    eye = lax.broadcasted_iota(jnp.int32, (NUM_SLOTS, NUM_SLOTS), 0) == \
        lax.broadcasted_iota(jnp.int32, (NUM_SLOTS, NUM_SLOTS), 1)
    rmat_ext = jnp.concatenate([rmat, jnp.where(eye, 0, -1)], axis=0)
    vals_ext = jnp.concatenate([vals, slots], axis=0)

    sel_rank = jnp.max(rmat_ext, axis=0, keepdims=True)
    wsel = jnp.where(rmat_ext == sel_rank, 1.0, 0.0)
    new_slots = lax.dot_general(wsel, vals_ext, (((0,), (0,)), ((), ())),
                                precision=_HIGH)

    out_ref[...] = jnp.dot(new_slots, wp_ref[...]) + bp_ref[...]


@jax.jit
def kernel(lm_tokens, W1, b1, W2, b2, slots, Wg, bg, Wp, bp):
    w2c = W2[:, :SLOT_DIM]
    wct = W2[:, SLOT_DIM:SLOT_DIM + 1].T
    b2c = b2[:SLOT_DIM].reshape(1, SLOT_DIM)
    bc = b2[SLOT_DIM:].reshape(1, 1)
    b1r = b1.reshape(1, HIDDEN)
    bgr = bg.reshape(1, SLOT_DIM)
    bpr = bp.reshape(1, LM_HIDDEN)

    half = B // 2

    def run_conf(x_half):
        return pl.pallas_call(
            _conf_kernel,
            grid=(half // FB,),
            in_specs=[
                pl.BlockSpec((FB, P, LM_HIDDEN), lambda b: (b, 0, 0)),
                pl.BlockSpec((LM_HIDDEN, HIDDEN), lambda b: (0, 0)),
                pl.BlockSpec((1, HIDDEN), lambda b: (0, 0)),
                pl.BlockSpec((1, HIDDEN), lambda b: (0, 0)),
                pl.BlockSpec((1, 1), lambda b: (0, 0)),
            ],
            out_specs=pl.BlockSpec((1, 1, FB * P), lambda b: (b, 0, 0)),
            out_shape=jax.ShapeDtypeStruct((half // FB, 1, FB * P),
                                           jnp.float32),
            compiler_params=pltpu.CompilerParams(
                dimension_semantics=("arbitrary",)),
        )(x_half, W1, b1r, wct, bc)

    x2 = lm_tokens.reshape(B * P, LM_HIDDEN)

    def run_sc(conf_half, base):
        sc_gather = pl.kernel(
            _make_sc_body(base, half),
            out_type=jax.ShapeDtypeStruct((half * MROWS, LM_HIDDEN),
                                          jnp.float32),
            mesh=plsc.VectorSubcoreMesh(core_axis_name="c",
                                        subcore_axis_name="s",
                                        num_cores=2, num_subcores=16),
            scratch_types=[
                pltpu.VMEM((P,), jnp.float32),
                pltpu.VMEM((LANES,), jnp.int32),
                pltpu.VMEM((LANES,), jnp.float32),
                pltpu.VMEM((LANES,), jnp.int32),
                pltpu.VMEM((MROWS, LM_HIDDEN), jnp.float32),
                pltpu.SemaphoreType.DMA,
            ],
            compiler_params=pltpu.CompilerParams(needs_layout_passes=False),
        )
        return sc_gather(conf_half.reshape(half, P), x2)

    conf_a = run_conf(lm_tokens[:half])
    xsel_a = run_sc(conf_a, 0)
    conf_b = run_conf(lm_tokens[half:])
    xsel_b = run_sc(conf_b, half)

    slot_lm = pl.pallas_call(
        _finish_kernel,
        out_shape=jax.ShapeDtypeStruct((NUM_SLOTS, LM_HIDDEN), jnp.float32),
    )(xsel_a, xsel_b, W1, b1r, w2c, b2c, slots, Wg, bgr, Wp, bpr)
    return slot_lm


# speedup vs baseline: 1.7262x; 1.7262x over previous
import math

import jax
import jax.numpy as jnp
from jax import lax
from jax.experimental import pallas as pl
from jax.experimental.pallas import tpu as pltpu
from jax.experimental.pallas import tpu_sc as plsc

NUM_SLOTS = 16
SLOT_DIM = 512
LM_HIDDEN = 2560
HIDDEN = 1024
TOP_M = 5
THR = 0.7
B = 32
P = 576
MROWS = 8
LANES = 16
NSEL = B * MROWS

_HIGH = jax.lax.Precision.HIGHEST
_INV_SQRT2 = 1.0 / math.sqrt(2.0)


def _gelu_exact(x):
    return 0.5 * x * (1.0 + lax.erf(x * _INV_SQRT2))


FB = 1


def _conf_kernel(x_ref, w1_ref, b1_ref, wct_ref, bc_ref, conf_ref):
    x = x_ref[...].reshape(FB * P, LM_HIDDEN)
    h = _gelu_exact(jnp.dot(x, w1_ref[...]) + b1_ref[...])
    conf = lax.dot_general(wct_ref[...], h, (((1,), (1,)), ((), ())))
    conf_ref[0] = conf + bc_ref[...]


def _combine(a, b):
    va, ga = a
    vb, gb = b
    keep_a = (va > vb) | ((va == vb) & (ga < gb))
    return jnp.where(keep_a, va, vb), jnp.where(keep_a, ga, gb)


def _sc_body(conf_hbm, x_hbm, xsel_hbm, conf_v, idx_v, bcf_v, bci_v, rows_v,
             sem):
    c = lax.axis_index("c")
    s = lax.axis_index("s")
    b = s * 2 + c
    pltpu.sync_copy(conf_hbm.at[b], conf_v)

    lanes = lax.iota(jnp.int32, 16)
    nchunks = P // LANES
    gidx = [ci * LANES + lanes for ci in range(nchunks)]
    vals = [conf_v[pl.ds(ci * LANES, LANES)] for ci in range(nchunks)]

    picked = []
    for m in range(TOP_M):
        if picked:
            last = picked[-1]
            vals = [jnp.where(g == last, -jnp.inf, v)
                    for g, v in zip(gidx, vals)]
        items = list(zip(vals, gidx))
        while len(items) > 1:
            nxt = [_combine(items[i], items[i + 1])
                   for i in range(0, len(items) - 1, 2)]
            if len(items) % 2:
                nxt.append(items[-1])
            items = nxt
        bv, bi = items[0]
        bcf_v[...] = bv
        bci_v[...] = bi
        splats = []
        for j in range(LANES):
            sel = jnp.full((LANES,), j, jnp.int32)
            splats.append((plsc.load_gather(bcf_v, [sel]),
                           plsc.load_gather(bci_v, [sel])))
        while len(splats) > 1:
            splats = [_combine(splats[i], splats[i + 1])
                      for i in range(0, len(splats), 2)]
        picked.append(splats[0][1])

    acc = jnp.zeros((LANES,), jnp.int32)
    for m, idx_m in enumerate(picked):
        acc = jnp.where(lanes == m, b * P + idx_m, acc)
    idx_v[...] = acc
    pltpu.async_copy(x_hbm.at[idx_v.at[pl.ds(0, MROWS)]], rows_v,
                     sem).wait()
    pltpu.sync_copy(rows_v, xsel_hbm.at[pl.ds(b * MROWS, MROWS)])


def _finish_kernel(xsel_ref, w1_ref, b1_ref, w2c_ref, b2c_ref, slots_ref,
                   wg_ref, bg_ref, wp_ref, bp_ref, out_ref):
    h = _gelu_exact(jnp.dot(xsel_ref[...], w1_ref[...]) + b1_ref[...])
    cand = jnp.dot(h, w2c_ref[...]) + b2c_ref[...]

    cn = cand / (jnp.sqrt(jnp.sum(cand * cand, axis=1, keepdims=True)) + 1e-12)
    slots = slots_ref[...]
    sn = slots / (jnp.sqrt(jnp.sum(slots * slots, axis=1, keepdims=True)) + 1e-12)
    scores = lax.dot_general(cn, sn, (((1,), (1,)), ((), ())))

    best_score = jnp.max(scores, axis=1, keepdims=True)
    s_iota = lax.broadcasted_iota(jnp.int32, (NSEL, NUM_SLOTS), 1)
    idx = jnp.min(jnp.where(scores == best_score, s_iota, NUM_SLOTS),
                  axis=1, keepdims=True)

    onehot = jnp.where(s_iota == idx, 1.0, 0.0)
    old = jnp.dot(onehot, slots, precision=_HIGH)

    g = jax.nn.sigmoid(
        jnp.dot(jnp.concatenate([old, cand], axis=1), wg_ref[...]) + bg_ref[...])
    upd = g * cand + (1.0 - g) * old
    vals = jnp.where(best_score > THR, upd, old)

    r_iota = lax.broadcasted_iota(jnp.int32, (NSEL, 1), 0)
    bv, mv = r_iota // MROWS, r_iota % MROWS
    valid = mv < TOP_M
    rank = jnp.where(valid, 1 + bv * TOP_M + mv, -1)
    rmat = jnp.where((s_iota == idx) & valid, rank, -1)

    eye = lax.broadcasted_iota(jnp.int32, (NUM_SLOTS, NUM_SLOTS), 0) == \
        lax.broadcasted_iota(jnp.int32, (NUM_SLOTS, NUM_SLOTS), 1)
    rmat_ext = jnp.concatenate([rmat, jnp.where(eye, 0, -1)], axis=0)
    vals_ext = jnp.concatenate([vals, slots], axis=0)

    sel_rank = jnp.max(rmat_ext, axis=0, keepdims=True)
    wsel = jnp.where(rmat_ext == sel_rank, 1.0, 0.0)
    new_slots = lax.dot_general(wsel, vals_ext, (((0,), (0,)), ((), ())),
                                precision=_HIGH)

    out_ref[...] = jnp.dot(new_slots, wp_ref[...]) + bp_ref[...]


@jax.jit
def kernel(lm_tokens, W1, b1, W2, b2, slots, Wg, bg, Wp, bp):
    w2c = W2[:, :SLOT_DIM]
    wct = W2[:, SLOT_DIM:SLOT_DIM + 1].T
    b2c = b2[:SLOT_DIM].reshape(1, SLOT_DIM)
    bc = b2[SLOT_DIM:].reshape(1, 1)
    b1r = b1.reshape(1, HIDDEN)
    bgr = bg.reshape(1, SLOT_DIM)
    bpr = bp.reshape(1, LM_HIDDEN)

    conf = pl.pallas_call(
        _conf_kernel,
        grid=(B // FB,),
        in_specs=[
            pl.BlockSpec((FB, P, LM_HIDDEN), lambda b: (b, 0, 0)),
            pl.BlockSpec((LM_HIDDEN, HIDDEN), lambda b: (0, 0)),
            pl.BlockSpec((1, HIDDEN), lambda b: (0, 0)),
            pl.BlockSpec((1, HIDDEN), lambda b: (0, 0)),
            pl.BlockSpec((1, 1), lambda b: (0, 0)),
        ],
        out_specs=pl.BlockSpec((1, 1, FB * P), lambda b: (b, 0, 0)),
        out_shape=jax.ShapeDtypeStruct((B // FB, 1, FB * P), jnp.float32),
        compiler_params=pltpu.CompilerParams(
            dimension_semantics=("arbitrary",)),
    )(lm_tokens, W1, b1r, wct, bc)

    conf2 = conf.reshape(B, P)
    x2 = lm_tokens.reshape(B * P, LM_HIDDEN)

    sc_gather = pl.kernel(
        _sc_body,
        out_type=jax.ShapeDtypeStruct((NSEL, LM_HIDDEN), jnp.float32),
        mesh=plsc.VectorSubcoreMesh(core_axis_name="c", subcore_axis_name="s",
                                    num_cores=2, num_subcores=16),
        scratch_types=[
            pltpu.VMEM((P,), jnp.float32),
            pltpu.VMEM((LANES,), jnp.int32),
            pltpu.VMEM((LANES,), jnp.float32),
            pltpu.VMEM((LANES,), jnp.int32),
            pltpu.VMEM((MROWS, LM_HIDDEN), jnp.float32),
            pltpu.SemaphoreType.DMA,
        ],
        compiler_params=pltpu.CompilerParams(needs_layout_passes=False),
    )
    xsel = sc_gather(conf2, x2)

    slot_lm = pl.pallas_call(
        _finish_kernel,
        out_shape=jax.ShapeDtypeStruct((NUM_SLOTS, LM_HIDDEN), jnp.float32),
    )(xsel, W1, b1r, w2c, b2c, slots, Wg, bgr, Wp, bpr)
    return slot_lm
